# speedup vs baseline: 7.3140x; 7.3140x over previous
import functools
import jax
import jax.numpy as jnp
from jax import lax
from jax.experimental import pallas as pl
from jax.experimental.pallas import tpu as pltpu
from jax.experimental.pallas import tpu_sc as plsc

N = 10000
E = 320000
D = 128
H = 128
B = 4096

NC = 2
NS = 16
NW = NC * NS
NP = 10240
ET = E // NW
C = 80
NCHUNK = ET // C
RPT = NP // NS

_mesh = plsc.VectorSubcoreMesh(
    core_axis_name="c", subcore_axis_name="s", num_cores=NC, num_subcores=NS
)


def _sc_agg_body(h_hbm, er_hbm, z2_hbm, z1_hbm, agg_out, deg_out,
                 src_idx, dst_idx, rows, ones, agg_s, deg_s, sem):
    cid = lax.axis_index("c")
    sid = lax.axis_index("s")
    wid = sid * NC + cid

    pltpu.sync_copy(z2_hbm.at[pl.ds(sid * RPT, RPT)],
                    agg_s.at[pl.ds(sid * RPT, RPT)])
    pltpu.sync_copy(z1_hbm.at[pl.ds(sid * RPT, RPT)],
                    deg_s.at[pl.ds(sid * RPT, RPT)])

    pltpu.sync_copy(er_hbm.at[0, wid], src_idx)
    pltpu.sync_copy(er_hbm.at[1, wid], dst_idx)

    for k in range(C // 16):
        ones[pl.ds(k * 16, 16)] = jnp.ones((16,), jnp.float32)

    plsc.subcore_barrier()

    def body(j, carry):
        pltpu.async_copy(h_hbm.at[src_idx.at[j]], rows, sem).wait()
        pltpu.sync_copy(rows, agg_s.at[dst_idx.at[j]], add=True)
        pltpu.sync_copy(ones, deg_s.at[dst_idx.at[j]], add=True)
        return carry

    lax.fori_loop(0, NCHUNK, body, 0)

    plsc.subcore_barrier()

    pltpu.sync_copy(agg_s.at[pl.ds(sid * RPT, RPT)],
                    agg_out.at[cid, pl.ds(sid * RPT, RPT)])
    pltpu.sync_copy(deg_s.at[pl.ds(sid * RPT, RPT)],
                    deg_out.at[cid, pl.ds(sid * RPT, RPT)])


_sc_agg = pl.kernel(
    _sc_agg_body,
    out_type=(
        jax.ShapeDtypeStruct((NC, NP, H), jnp.float32),
        jax.ShapeDtypeStruct((NC, NP), jnp.float32),
    ),
    mesh=_mesh,
    scratch_types=[
        pltpu.VMEM((NCHUNK, C), jnp.int32),
        pltpu.VMEM((NCHUNK, C), jnp.int32),
        pltpu.VMEM((C, H), jnp.float32),
        pltpu.VMEM((C,), jnp.float32),
        pltpu.VMEM_SHARED((NP, H), jnp.float32),
        pltpu.VMEM_SHARED((NP,), jnp.float32),
        pltpu.SemaphoreType.DMA,
    ],
)


def _sc_pair_body(h_hbm, pidx_hbm, out_hbm, idx_v, rows, sem):
    cid = lax.axis_index("c")
    sid = lax.axis_index("s")
    wid = sid * NC + cid
    pltpu.sync_copy(pidx_hbm.at[wid], idx_v)
    for j in range(2):
        pltpu.async_copy(h_hbm.at[idx_v.at[j]], rows, sem).wait()
        pltpu.sync_copy(rows, out_hbm.at[pl.ds(wid * 256 + j * 128, 128)])


_sc_pair = pl.kernel(
    _sc_pair_body,
    out_type=jax.ShapeDtypeStruct((2 * B, H), jnp.float32),
    mesh=_mesh,
    scratch_types=[
        pltpu.VMEM((2, 128), jnp.int32),
        pltpu.VMEM((128, H), jnp.float32),
        pltpu.SemaphoreType.DMA,
    ],
)


def _tc_layer_body(agg, invd, h, wl, bl, wr, out, *, relu):
    mean = (agg[0] + agg[1]) * invd[...]
    z = (jnp.dot(mean, wl[...], preferred_element_type=jnp.float32)
         + jnp.dot(h[...], wr[...], preferred_element_type=jnp.float32)
         + bl[...])
    out[...] = jnp.maximum(z, 0.0) if relu else z


def _tc_layer(agg, invd, h, wl, bl, wr, relu):
    rb = 512
    grid = NP // rb
    return pl.pallas_call(
        functools.partial(_tc_layer_body, relu=relu),
        grid=(grid,),
        in_specs=[
            pl.BlockSpec((NC, rb, H), lambda i: (0, i, 0)),
            pl.BlockSpec((rb, H), lambda i: (i, 0)),
            pl.BlockSpec((rb, H), lambda i: (i, 0)),
            pl.BlockSpec((H, H), lambda i: (0, 0)),
            pl.BlockSpec((1, H), lambda i: (0, 0)),
            pl.BlockSpec((H, H), lambda i: (0, 0)),
        ],
        out_specs=pl.BlockSpec((rb, H), lambda i: (i, 0)),
        out_shape=jax.ShapeDtypeStruct((NP, H), jnp.float32),
    )(agg, invd, h, wl, bl, wr)


def _tc_mlp_body(se, te, ef, wew, web, w1a, w1b, w1c, b1, g1s, be1,
                 w2, b2, w3, b3, out):
    ep = jnp.maximum(
        jnp.dot(ef[...], wew[...], preferred_element_type=jnp.float32)
        + web[...], 0.0)
    z = (jnp.dot(se[...], w1a[...], preferred_element_type=jnp.float32)
         + jnp.dot(te[...], w1b[...], preferred_element_type=jnp.float32)
         + jnp.dot(ep, w1c[...], preferred_element_type=jnp.float32)
         + b1[...])
    z = jnp.maximum(z, 0.0) * g1s[...] + be1[...]
    z = jnp.maximum(
        jnp.dot(z, w2[...], preferred_element_type=jnp.float32) + b2[...],
        0.0)
    o = jnp.dot(z, w3[...], preferred_element_type=jnp.float32) + b3[...]
    out[...] = jax.nn.sigmoid(o)


def _tc_mlp(pe, efp, wewp, web, w1a, w1b, w1c, b1, g1s, be1, w2p, b2p,
            w3p, b3):
    rb = 512
    grid = B // rb
    full = lambda r, c: pl.BlockSpec((r, c), lambda i: (0, 0))
    return pl.pallas_call(
        _tc_mlp_body,
        grid=(grid,),
        in_specs=[
            pl.BlockSpec((rb, H), lambda i: (i, 0)),
            pl.BlockSpec((rb, H), lambda i: (i + B // rb, 0)),
            pl.BlockSpec((rb, 128), lambda i: (i, 0)),
            full(128, 64),
            full(1, 64),
            full(H, H),
            full(H, H),
            full(64, H),
            full(1, H),
            full(1, H),
            full(1, H),
            full(H, 128),
            full(1, 128),
            full(128, 1),
            full(1, 1),
        ],
        out_specs=pl.BlockSpec((rb, 1), lambda i: (i, 0)),
        out_shape=jax.ShapeDtypeStruct((B, 1), jnp.float32),
    )(pe, pe, efp, wewp, web, w1a, w1b, w1c, b1, g1s, be1, w2p, b2p,
      w3p, b3)


def kernel(x, edge_index, src_nodes, tgt_nodes, edge_features, Wl0, bl0,
           Wr0, Wl1, bl1, Wr1, We_w, We_b, W1, b1, g1, be1, W2, b2, W3,
           b3):
    er = edge_index.reshape(2, NW, NCHUNK, C)
    xp = jnp.pad(x, ((0, NP - N), (0, 0)))
    z2 = jnp.zeros((NP, H), jnp.float32)
    z1 = jnp.zeros((NP,), jnp.float32)

    agg0, deg = _sc_agg(xp, er, z2, z1)
    invd = jnp.broadcast_to(
        (1.0 / jnp.maximum(deg[0] + deg[1], 1.0))[:, None], (NP, H))
    h1 = _tc_layer(agg0, invd, xp, Wl0, bl0.reshape(1, H), Wr0, True)

    agg1, _ = _sc_agg(h1, er, z2, z1)
    h2 = _tc_layer(agg1, invd, h1, Wl1, bl1.reshape(1, H), Wr1, False)

    pidx = jnp.concatenate([src_nodes, tgt_nodes]).reshape(NW, 2, 128)
    pe = _sc_pair(h2, pidx)

    efp = jnp.pad(edge_features, ((0, 0), (0, 128 - edge_features.shape[1])))
    wewp = jnp.pad(We_w, ((0, 128 - We_w.shape[0]), (0, 0)))
    w1a, w1b, w1c = W1[:H], W1[H:2 * H], W1[2 * H:]
    g1s = (g1 / jnp.sqrt(1.0 + 1e-5)).reshape(1, H)
    w2p = jnp.pad(W2, ((0, 0), (0, 128 - W2.shape[1])))
    b2p = jnp.pad(b2, (0, 128 - b2.shape[0])).reshape(1, 128)
    w3p = jnp.pad(W3, ((0, 128 - W3.shape[0]), (0, 0)))

    return _tc_mlp(pe, efp, wewp, We_b.reshape(1, 64), w1a, w1b, w1c,
                   b1.reshape(1, H), g1s, be1.reshape(1, H), w2p, b2p,
                   w3p, b3.reshape(1, 1))
